# baseline (speedup 1.0000x reference)
import functools

import jax
import jax.numpy as jnp
from jax import lax
from jax.experimental import pallas as pl
from jax.experimental.pallas import tpu as pltpu

EPS = 1e-5


def _round_up(x, m):
    return (x + m - 1) // m * m


def _gram_kernel(feat_ref, c_ref, rs_ref):
    i = pl.program_id(0)

    @pl.when(i == 0)
    def _():
        c_ref[...] = jnp.zeros_like(c_ref)
        rs_ref[...] = jnp.zeros_like(rs_ref)

    fb = feat_ref[...].astype(jnp.bfloat16)
    c_ref[...] += lax.dot_general(
        fb, fb, (((0,), (0,)), ((), ())),
        preferred_element_type=jnp.float32)
    tm, f_sz = feat_ref.shape
    rs_ref[...] += jnp.sum(
        fb.astype(jnp.float32).reshape(tm // 8, 8, f_sz), axis=0)


def _stats_kernel(c_ref, rs_ref, w1t_ref, p_ref, ap_ref, *, n_rows):
    inv_n = 1.0 / n_rows
    w1t = w1t_ref[...].astype(jnp.float32)
    d = jnp.dot(c_ref[...], w1t, preferred_element_type=jnp.float32)
    e2 = jnp.sum(w1t * d, axis=0, keepdims=True) * inv_n
    m8 = jnp.dot(rs_ref[...], w1t, preferred_element_type=jnp.float32)
    mean = jnp.sum(m8, axis=0, keepdims=True) * inv_n
    var = e2 - mean * mean
    inv_std = lax.rsqrt(jnp.maximum(var, 0.0) + EPS)
    a = inv_std * p_ref[1:2, :]
    ap_ref[...] = p_ref[...]
    ap_ref[1:2, :] = a
    ap_ref[2:3, :] = p_ref[2:3, :] - mean * a


def _apply_kernel(feat_ref, w1t_ref, wc_ref, p_ref, out_ref):
    units = wc_ref.shape[1]

    bias = p_ref[0:1, :]
    bn_a = p_ref[1:2, :]
    bn_b = p_ref[2:3, :]
    ln_g = p_ref[3:4, :]
    ln_b = p_ref[4:5, :]

    fb = feat_ref[...].astype(jnp.bfloat16)
    h = jnp.dot(fb, w1t_ref[...], preferred_element_type=jnp.float32)

    h = h * bn_a + bn_b
    hb = jnp.maximum(h, 0.0).astype(jnp.bfloat16)

    lhs = jnp.concatenate([hb, fb], axis=1)
    f = (jnp.dot(lhs, wc_ref[...], preferred_element_type=jnp.float32)
         + bias)

    inv_u = 1.0 / units
    mu = jnp.sum(f, axis=-1, keepdims=True) * inv_u
    ef2 = jnp.sum(f * f, axis=-1, keepdims=True) * inv_u
    v = jnp.maximum(ef2 - mu * mu, 0.0)
    c = lax.rsqrt(v + EPS) * ln_g
    out_ref[...] = (f * c + (ln_b - mu * c)).astype(out_ref.dtype)


def kernel(feat, w1, w2, b2, ws, bs, bn_gamma, bn_beta, ln_gamma, ln_beta):
    n, f_sz = feat.shape
    u = w2.shape[0]

    w1t = w1.T.astype(jnp.bfloat16)
    wcat = jnp.concatenate([w2.T, ws.T], axis=0).astype(jnp.bfloat16)
    pvec = jnp.zeros((8, u), jnp.float32)
    pvec = pvec.at[0].set(b2 + bs)
    pvec = pvec.at[1].set(bn_gamma)
    pvec = pvec.at[2].set(bn_beta)
    pvec = pvec.at[3].set(ln_gamma)
    pvec = pvec.at[4].set(ln_beta)

    tm = 2048
    tm1 = 2048
    vmem_limit = 58 * 1024 * 1024
    n_pad = _round_up(n, max(tm, tm1))
    feat_p = jnp.pad(feat, ((0, n_pad - n), (0, 0))) if n_pad != n else feat
    n1 = n_pad // tm
    ng = n_pad // tm1

    c_acc, rs_acc = pl.pallas_call(
        _gram_kernel,
        out_shape=(
            jax.ShapeDtypeStruct((f_sz, f_sz), jnp.float32),
            jax.ShapeDtypeStruct((8, f_sz), jnp.float32),
        ),
        grid=(ng,),
        in_specs=[
            pl.BlockSpec((tm1, f_sz), lambda i: (i, 0)),
        ],
        out_specs=(
            pl.BlockSpec((f_sz, f_sz), lambda i: (0, 0)),
            pl.BlockSpec((8, f_sz), lambda i: (0, 0)),
        ),
        compiler_params=pltpu.CompilerParams(
            dimension_semantics=("arbitrary",),
            vmem_limit_bytes=vmem_limit),
    )(feat_p)

    papp = pl.pallas_call(
        functools.partial(_stats_kernel, n_rows=float(n)),
        out_shape=jax.ShapeDtypeStruct((8, u), jnp.float32),
        grid=(1,),
        in_specs=[
            pl.BlockSpec((f_sz, f_sz), lambda i: (0, 0)),
            pl.BlockSpec((8, f_sz), lambda i: (0, 0)),
            pl.BlockSpec((f_sz, u), lambda i: (0, 0)),
            pl.BlockSpec((8, u), lambda i: (0, 0)),
        ],
        out_specs=pl.BlockSpec((8, u), lambda i: (0, 0)),
        compiler_params=pltpu.CompilerParams(
            dimension_semantics=("arbitrary",),
            vmem_limit_bytes=vmem_limit),
    )(c_acc, rs_acc, w1t, pvec)

    out = pl.pallas_call(
        _apply_kernel,
        out_shape=jax.ShapeDtypeStruct((n_pad, u), feat.dtype),
        grid=(n1,),
        in_specs=[
            pl.BlockSpec((tm, f_sz), lambda i: (i, 0)),
            pl.BlockSpec((f_sz, u), lambda i: (0, 0)),
            pl.BlockSpec((u + f_sz, u), lambda i: (0, 0)),
            pl.BlockSpec((8, u), lambda i: (0, 0)),
        ],
        out_specs=pl.BlockSpec((tm, u), lambda i: (i, 0)),
        compiler_params=pltpu.CompilerParams(
            dimension_semantics=("arbitrary",),
            vmem_limit_bytes=vmem_limit),
    )(feat_p, w1t, wcat, papp)

    return out[:n] if n_pad != n else out


# speedup vs baseline: 1.0114x; 1.0114x over previous
import functools

import jax
import jax.numpy as jnp
from jax import lax
from jax.experimental import pallas as pl
from jax.experimental.pallas import tpu as pltpu

EPS = 1e-5


def _round_up(x, m):
    return (x + m - 1) // m * m


def _gram_kernel(feat_ref, c_ref, rs_ref):
    i = pl.program_id(0)

    @pl.when(i == 0)
    def _():
        c_ref[...] = jnp.zeros_like(c_ref)
        rs_ref[...] = jnp.zeros_like(rs_ref)

    fb = feat_ref[...].astype(jnp.bfloat16)
    c_ref[...] += lax.dot_general(
        fb, fb, (((0,), (0,)), ((), ())),
        preferred_element_type=jnp.float32)
    tm, f_sz = feat_ref.shape
    rs_ref[...] += jnp.sum(
        fb.astype(jnp.float32).reshape(tm // 8, 8, f_sz), axis=0)


def _stats_kernel(c_ref, rs_ref, w1t_ref, p_ref, ap_ref, *, n_rows):
    inv_n = 1.0 / n_rows
    w1t = w1t_ref[...].astype(jnp.float32)
    d = jnp.dot(c_ref[...], w1t, preferred_element_type=jnp.float32)
    e2 = jnp.sum(w1t * d, axis=0, keepdims=True) * inv_n
    m8 = jnp.dot(rs_ref[...], w1t, preferred_element_type=jnp.float32)
    mean = jnp.sum(m8, axis=0, keepdims=True) * inv_n
    var = e2 - mean * mean
    inv_std = lax.rsqrt(jnp.maximum(var, 0.0) + EPS)
    a = inv_std * p_ref[1:2, :]
    ap_ref[...] = p_ref[...]
    ap_ref[1:2, :] = a
    ap_ref[2:3, :] = p_ref[2:3, :] - mean * a


def _apply_kernel(feat_ref, w1t_ref, wc_ref, p_ref, out_ref):
    units = wc_ref.shape[1]

    bias = p_ref[0:1, :]
    bn_a = p_ref[1:2, :]
    bn_b = p_ref[2:3, :]
    ln_g = p_ref[3:4, :]
    ln_b = p_ref[4:5, :]

    fb = feat_ref[...].astype(jnp.bfloat16)
    h = jnp.dot(fb, w1t_ref[...], preferred_element_type=jnp.float32)

    h = h * bn_a + bn_b
    hb = jnp.maximum(h, 0.0).astype(jnp.bfloat16)

    lhs = jnp.concatenate([hb, fb], axis=1)
    f = (jnp.dot(lhs, wc_ref[...], preferred_element_type=jnp.float32)
         + bias)

    inv_u = 1.0 / units
    mu = jnp.sum(f, axis=-1, keepdims=True) * inv_u
    ef2 = jnp.sum(f * f, axis=-1, keepdims=True) * inv_u
    v = jnp.maximum(ef2 - mu * mu, 0.0)
    c = lax.rsqrt(v + EPS) * ln_g
    out_ref[...] = (f * c + (ln_b - mu * c)).astype(out_ref.dtype)


def kernel(feat, w1, w2, b2, ws, bs, bn_gamma, bn_beta, ln_gamma, ln_beta):
    n, f_sz = feat.shape
    u = w2.shape[0]

    w1t = w1.T.astype(jnp.bfloat16)
    wcat = jnp.concatenate([w2.T, ws.T], axis=0).astype(jnp.bfloat16)
    pvec = jnp.zeros((8, u), jnp.float32)
    pvec = pvec.at[0].set(b2 + bs)
    pvec = pvec.at[1].set(bn_gamma)
    pvec = pvec.at[2].set(bn_beta)
    pvec = pvec.at[3].set(ln_gamma)
    pvec = pvec.at[4].set(ln_beta)

    tm = 1024
    tm1 = 2048
    vmem_limit = 48 * 1024 * 1024
    n_pad = _round_up(n, max(tm, tm1))
    feat_p = jnp.pad(feat, ((0, n_pad - n), (0, 0))) if n_pad != n else feat
    n1 = n_pad // tm
    ng = n_pad // tm1

    c_acc, rs_acc = pl.pallas_call(
        _gram_kernel,
        out_shape=(
            jax.ShapeDtypeStruct((f_sz, f_sz), jnp.float32),
            jax.ShapeDtypeStruct((8, f_sz), jnp.float32),
        ),
        grid=(ng,),
        in_specs=[
            pl.BlockSpec((tm1, f_sz), lambda i: (i, 0)),
        ],
        out_specs=(
            pl.BlockSpec((f_sz, f_sz), lambda i: (0, 0)),
            pl.BlockSpec((8, f_sz), lambda i: (0, 0)),
        ),
        compiler_params=pltpu.CompilerParams(
            dimension_semantics=("arbitrary",),
            vmem_limit_bytes=vmem_limit),
    )(feat_p)

    papp = pl.pallas_call(
        functools.partial(_stats_kernel, n_rows=float(n)),
        out_shape=jax.ShapeDtypeStruct((8, u), jnp.float32),
        grid=(1,),
        in_specs=[
            pl.BlockSpec((f_sz, f_sz), lambda i: (0, 0)),
            pl.BlockSpec((8, f_sz), lambda i: (0, 0)),
            pl.BlockSpec((f_sz, u), lambda i: (0, 0)),
            pl.BlockSpec((8, u), lambda i: (0, 0)),
        ],
        out_specs=pl.BlockSpec((8, u), lambda i: (0, 0)),
        compiler_params=pltpu.CompilerParams(
            dimension_semantics=("arbitrary",),
            vmem_limit_bytes=vmem_limit),
    )(c_acc, rs_acc, w1t, pvec)

    out = pl.pallas_call(
        _apply_kernel,
        out_shape=jax.ShapeDtypeStruct((n_pad, u), feat.dtype),
        grid=(n1,),
        in_specs=[
            pl.BlockSpec((tm, f_sz), lambda i: (i, 0)),
            pl.BlockSpec((f_sz, u), lambda i: (0, 0)),
            pl.BlockSpec((u + f_sz, u), lambda i: (0, 0)),
            pl.BlockSpec((8, u), lambda i: (0, 0)),
        ],
        out_specs=pl.BlockSpec((tm, u), lambda i: (i, 0)),
        compiler_params=pltpu.CompilerParams(
            dimension_semantics=("arbitrary",),
            allow_input_fusion=[False, True, True, False],
            vmem_limit_bytes=vmem_limit),
    )(feat_p, w1t, wcat, papp)

    return out[:n] if n_pad != n else out


# speedup vs baseline: 1.1115x; 1.0991x over previous
import functools

import jax
import jax.numpy as jnp
from jax import lax
from jax.experimental import pallas as pl
from jax.experimental.pallas import tpu as pltpu

EPS = 1e-5


def _round_up(x, m):
    return (x + m - 1) // m * m


def _gram_kernel(feat_ref, c_ref, rs_ref):
    i = pl.program_id(0)

    @pl.when(i == 0)
    def _():
        c_ref[...] = jnp.zeros_like(c_ref)
        rs_ref[...] = jnp.zeros_like(rs_ref)

    fb = feat_ref[...].astype(jnp.bfloat16)
    c_ref[...] += lax.dot_general(
        fb, fb, (((0,), (0,)), ((), ())),
        preferred_element_type=jnp.float32)
    tm, f_sz = feat_ref.shape
    rs_ref[...] += jnp.sum(
        fb.astype(jnp.float32).reshape(tm // 8, 8, f_sz), axis=0)


def _stats_kernel(c_ref, rs_ref, w1t_ref, p_ref, ap_ref, *, n_rows):
    inv_n = 1.0 / n_rows
    w1t = w1t_ref[...].astype(jnp.float32)
    d = jnp.dot(c_ref[...], w1t, preferred_element_type=jnp.float32)
    e2 = jnp.sum(w1t * d, axis=0, keepdims=True) * inv_n
    m8 = jnp.dot(rs_ref[...], w1t, preferred_element_type=jnp.float32)
    mean = jnp.sum(m8, axis=0, keepdims=True) * inv_n
    var = e2 - mean * mean
    inv_std = lax.rsqrt(jnp.maximum(var, 0.0) + EPS)
    a = inv_std * p_ref[1:2, :]
    ap_ref[...] = p_ref[...]
    ap_ref[1:2, :] = a
    ap_ref[2:3, :] = p_ref[2:3, :] - mean * a


def _apply_kernel(feat_ref, w1t_ref, wc_ref, p_ref, out_ref):
    units = wc_ref.shape[1]

    bias = p_ref[0:1, :]
    bn_a = p_ref[1:2, :]
    bn_b = p_ref[2:3, :]
    ln_g = p_ref[3:4, :]
    ln_b = p_ref[4:5, :]

    fb = feat_ref[...].astype(jnp.bfloat16)
    h = jnp.dot(fb, w1t_ref[...], preferred_element_type=jnp.float32)

    h = h * bn_a + bn_b
    hb = jnp.maximum(h, 0.0).astype(jnp.bfloat16)

    lhs = jnp.concatenate([hb, fb], axis=1)
    f = (jnp.dot(lhs, wc_ref[...], preferred_element_type=jnp.float32)
         + bias)

    inv_u = 1.0 / units
    mu = jnp.sum(f, axis=-1, keepdims=True) * inv_u
    ef2 = jnp.sum(f * f, axis=-1, keepdims=True) * inv_u
    v = jnp.maximum(ef2 - mu * mu, 0.0)
    c = lax.rsqrt(v + EPS) * ln_g
    out_ref[...] = (f * c + (ln_b - mu * c)).astype(out_ref.dtype)


def kernel(feat, w1, w2, b2, ws, bs, bn_gamma, bn_beta, ln_gamma, ln_beta):
    n, f_sz = feat.shape
    u = w2.shape[0]

    w1t = w1.T.astype(jnp.bfloat16)
    wcat = jnp.concatenate([w2.T, ws.T], axis=0).astype(jnp.bfloat16)
    pvec = jnp.zeros((8, u), jnp.float32)
    pvec = pvec.at[0].set(b2 + bs)
    pvec = pvec.at[1].set(bn_gamma)
    pvec = pvec.at[2].set(bn_beta)
    pvec = pvec.at[3].set(ln_gamma)
    pvec = pvec.at[4].set(ln_beta)

    tm = 1024
    tm1 = 4096
    vmem_limit = 48 * 1024 * 1024
    n_pad = _round_up(n, max(tm, tm1))
    feat_p = jnp.pad(feat, ((0, n_pad - n), (0, 0))) if n_pad != n else feat
    n1 = n_pad // tm
    ng = n_pad // tm1

    c_acc, rs_acc = pl.pallas_call(
        _gram_kernel,
        out_shape=(
            jax.ShapeDtypeStruct((f_sz, f_sz), jnp.float32),
            jax.ShapeDtypeStruct((8, f_sz), jnp.float32),
        ),
        grid=(ng,),
        in_specs=[
            pl.BlockSpec((tm1, f_sz), lambda i: (i, 0)),
        ],
        out_specs=(
            pl.BlockSpec((f_sz, f_sz), lambda i: (0, 0)),
            pl.BlockSpec((8, f_sz), lambda i: (0, 0)),
        ),
        compiler_params=pltpu.CompilerParams(
            dimension_semantics=("arbitrary",),
            vmem_limit_bytes=vmem_limit),
    )(feat_p)

    papp = pl.pallas_call(
        functools.partial(_stats_kernel, n_rows=float(n)),
        out_shape=jax.ShapeDtypeStruct((8, u), jnp.float32),
        grid=(1,),
        in_specs=[
            pl.BlockSpec((f_sz, f_sz), lambda i: (0, 0)),
            pl.BlockSpec((8, f_sz), lambda i: (0, 0)),
            pl.BlockSpec((f_sz, u), lambda i: (0, 0)),
            pl.BlockSpec((8, u), lambda i: (0, 0)),
        ],
        out_specs=pl.BlockSpec((8, u), lambda i: (0, 0)),
        compiler_params=pltpu.CompilerParams(
            dimension_semantics=("arbitrary",),
            vmem_limit_bytes=vmem_limit),
    )(c_acc, rs_acc, w1t, pvec)

    out = pl.pallas_call(
        _apply_kernel,
        out_shape=jax.ShapeDtypeStruct((n_pad, u), feat.dtype),
        grid=(n1,),
        in_specs=[
            pl.BlockSpec((tm, f_sz), lambda i: (i, 0)),
            pl.BlockSpec((f_sz, u), lambda i: (0, 0)),
            pl.BlockSpec((u + f_sz, u), lambda i: (0, 0)),
            pl.BlockSpec((8, u), lambda i: (0, 0)),
        ],
        out_specs=pl.BlockSpec((tm, u), lambda i: (i, 0)),
        compiler_params=pltpu.CompilerParams(
            dimension_semantics=("arbitrary",),
            vmem_limit_bytes=vmem_limit),
    )(feat_p, w1t, wcat, papp)

    return out[:n] if n_pad != n else out


# speedup vs baseline: 1.1127x; 1.0011x over previous
import functools

import jax
import jax.numpy as jnp
from jax import lax
from jax.experimental import pallas as pl
from jax.experimental.pallas import tpu as pltpu

EPS = 1e-5


def _round_up(x, m):
    return (x + m - 1) // m * m


def _gram_kernel(feat_ref, c_ref, rs_ref):
    i = pl.program_id(0)

    @pl.when(i == 0)
    def _():
        c_ref[...] = jnp.zeros_like(c_ref)
        rs_ref[...] = jnp.zeros_like(rs_ref)

    fb = feat_ref[...].astype(jnp.bfloat16)
    c_ref[...] += lax.dot_general(
        fb, fb, (((0,), (0,)), ((), ())),
        preferred_element_type=jnp.float32)
    tm, f_sz = feat_ref.shape
    rs_ref[...] += jnp.sum(
        fb.astype(jnp.float32).reshape(tm // 8, 8, f_sz), axis=0)


def _stats_kernel(c_ref, rs_ref, w1t_ref, p_ref, ap_ref, *, n_rows):
    inv_n = 1.0 / n_rows
    w1t = w1t_ref[...].astype(jnp.float32)
    d = jnp.dot(c_ref[...], w1t, preferred_element_type=jnp.float32)
    e2 = jnp.sum(w1t * d, axis=0, keepdims=True) * inv_n
    m8 = jnp.dot(rs_ref[...], w1t, preferred_element_type=jnp.float32)
    mean = jnp.sum(m8, axis=0, keepdims=True) * inv_n
    var = e2 - mean * mean
    inv_std = lax.rsqrt(jnp.maximum(var, 0.0) + EPS)
    a = inv_std * p_ref[1:2, :]
    ap_ref[...] = p_ref[...]
    ap_ref[1:2, :] = a
    ap_ref[2:3, :] = p_ref[2:3, :] - mean * a


def _apply_kernel(feat_ref, w1t_ref, wc_ref, p_ref, out_ref):
    units = wc_ref.shape[1]

    bias = p_ref[0:1, :]
    bn_a = p_ref[1:2, :]
    bn_b = p_ref[2:3, :]
    ln_g = p_ref[3:4, :]
    ln_b = p_ref[4:5, :]

    fb = feat_ref[...].astype(jnp.bfloat16)
    h = jnp.dot(fb, w1t_ref[...], preferred_element_type=jnp.float32)

    h = h * bn_a + bn_b
    hb = jnp.maximum(h, 0.0).astype(jnp.bfloat16)

    lhs = jnp.concatenate([hb, fb], axis=1)
    f = (jnp.dot(lhs, wc_ref[...], preferred_element_type=jnp.float32)
         + bias)

    inv_u = 1.0 / units
    mu = jnp.sum(f, axis=-1, keepdims=True) * inv_u
    ef2 = jnp.sum(f * f, axis=-1, keepdims=True) * inv_u
    v = jnp.maximum(ef2 - mu * mu, 0.0)
    c = lax.rsqrt(v + EPS) * ln_g
    out_ref[...] = (f * c + (ln_b - mu * c)).astype(out_ref.dtype)


def kernel(feat, w1, w2, b2, ws, bs, bn_gamma, bn_beta, ln_gamma, ln_beta):
    n, f_sz = feat.shape
    u = w2.shape[0]

    w1t = w1.T.astype(jnp.bfloat16)
    wcat = jnp.concatenate([w2.T, ws.T], axis=0).astype(jnp.bfloat16)
    pvec = jnp.zeros((8, u), jnp.float32)
    pvec = pvec.at[0].set(b2 + bs)
    pvec = pvec.at[1].set(bn_gamma)
    pvec = pvec.at[2].set(bn_beta)
    pvec = pvec.at[3].set(ln_gamma)
    pvec = pvec.at[4].set(ln_beta)

    tm = 1024
    tm1 = 8192
    vmem_limit = 48 * 1024 * 1024
    n_pad = _round_up(n, max(tm, tm1))
    feat_p = jnp.pad(feat, ((0, n_pad - n), (0, 0))) if n_pad != n else feat
    n1 = n_pad // tm
    ng = n_pad // tm1

    c_acc, rs_acc = pl.pallas_call(
        _gram_kernel,
        out_shape=(
            jax.ShapeDtypeStruct((f_sz, f_sz), jnp.float32),
            jax.ShapeDtypeStruct((8, f_sz), jnp.float32),
        ),
        grid=(ng,),
        in_specs=[
            pl.BlockSpec((tm1, f_sz), lambda i: (i, 0)),
        ],
        out_specs=(
            pl.BlockSpec((f_sz, f_sz), lambda i: (0, 0)),
            pl.BlockSpec((8, f_sz), lambda i: (0, 0)),
        ),
        compiler_params=pltpu.CompilerParams(
            dimension_semantics=("arbitrary",),
            vmem_limit_bytes=vmem_limit),
    )(feat_p)

    papp = pl.pallas_call(
        functools.partial(_stats_kernel, n_rows=float(n)),
        out_shape=jax.ShapeDtypeStruct((8, u), jnp.float32),
        grid=(1,),
        in_specs=[
            pl.BlockSpec((f_sz, f_sz), lambda i: (0, 0)),
            pl.BlockSpec((8, f_sz), lambda i: (0, 0)),
            pl.BlockSpec((f_sz, u), lambda i: (0, 0)),
            pl.BlockSpec((8, u), lambda i: (0, 0)),
        ],
        out_specs=pl.BlockSpec((8, u), lambda i: (0, 0)),
        compiler_params=pltpu.CompilerParams(
            dimension_semantics=("arbitrary",),
            vmem_limit_bytes=vmem_limit),
    )(c_acc, rs_acc, w1t, pvec)

    out = pl.pallas_call(
        _apply_kernel,
        out_shape=jax.ShapeDtypeStruct((n_pad, u), feat.dtype),
        grid=(n1,),
        in_specs=[
            pl.BlockSpec((tm, f_sz), lambda i: (i, 0)),
            pl.BlockSpec((f_sz, u), lambda i: (0, 0)),
            pl.BlockSpec((u + f_sz, u), lambda i: (0, 0)),
            pl.BlockSpec((8, u), lambda i: (0, 0)),
        ],
        out_specs=pl.BlockSpec((tm, u), lambda i: (i, 0)),
        compiler_params=pltpu.CompilerParams(
            dimension_semantics=("arbitrary",),
            vmem_limit_bytes=vmem_limit),
    )(feat_p, w1t, wcat, papp)

    return out[:n] if n_pad != n else out
